# baseline (speedup 1.0000x reference)
import functools
import numpy as np
import jax
import jax.numpy as jnp
from jax import lax
from jax.experimental import pallas as pl
from jax.experimental.pallas import tpu as pltpu, tpu_sc as plsc

NEURONS_C = 32768
K_C = 64
BATCH_C = 128
NWORKERS = 32
ROWS_PER_WORKER = BATCH_C // NWORKERS
NV_ROW = NEURONS_C // 16

MIN32 = np.int32(-2**31)
M7F = np.int32(0x7FFFFFFF)


def _skey(x):
    i = lax.bitcast_convert_type(x, jnp.int32)
    return i ^ (lax.shift_right_arithmetic(i, 31) & M7F)


def _sc_body(s_hbm, o_hbm, row_v, hist_v, suf_v, ck_v, ci_v):
    wid = lax.axis_index("s") * 2 + lax.axis_index("c")
    lane = lax.iota(jnp.int32, 16)
    lane_base = lane * 256
    ones16 = jnp.ones((16,), jnp.int32)
    zeros16 = jnp.zeros((16,), jnp.int32)

    def do_row(row):
        pltpu.sync_copy(s_hbm.at[row], row_v)

        def zh(i, _):
            hist_v[pl.ds(i * 16, 16)] = zeros16
            return 0
        lax.fori_loop(0, 256, zh, 0)

        def h(i, _):
            sk = _skey(row_v[pl.ds(i * 16, 16)])
            bin_ = lax.shift_right_logical(sk ^ MIN32, 24)
            plsc.addupdate_scatter(hist_v, [lane_base + bin_], ones16)
            return 0
        lax.fori_loop(0, NV_ROW, h, 0)

        suf_v[pl.ds(256, 16)] = zeros16
        running = jnp.int32(0)
        for c in range(15, -1, -1):
            tot = hist_v[pl.ds(c * 16, 16)]
            for j in range(1, 16):
                tot = tot + hist_v[pl.ds(j * 256 + c * 16, 16)]
            within = lax.rev(plsc.cumsum(lax.rev(tot, (0,))), (0,))
            suf_v[pl.ds(c * 16, 16)] = within + running
            running = running + jnp.sum(tot)

        b0 = zeros16
        for b in (128, 64, 32, 16, 8, 4, 2, 1):
            cand = b0 | np.int32(b)
            b0 = jnp.where(plsc.load_gather(suf_v, [cand]) >= K_C, cand, b0)
        c_above = plsc.load_gather(suf_v, [b0 + 1])
        k_rem = K_C - c_above
        n_cand = plsc.load_gather(suf_v, [b0]) - c_above

        def cp(i, off):
            sk = _skey(row_v[pl.ds(i * 16, 16)])
            bin_ = lax.shift_right_logical(sk ^ MIN32, 24)
            m = bin_ == b0
            incl = plsc.cumsum(m.astype(jnp.int32))
            pos = off + incl - 1
            plsc.store_scatter(ck_v, [pos], sk, mask=m)
            plsc.store_scatter(ci_v, [pos], lane + i * 16, mask=m)
            return off + plsc.all_reduce_population_count(m)
        off = lax.fori_loop(0, NV_ROW, cp, zeros16)
        plsc.store_scatter(ck_v, [off + lane], jnp.full((16,), MIN32))
        plsc.store_scatter(ci_v, [off + lane], zeros16)
        nv = ((n_cand + 15) >> 4)[0]

        def count_ge(t_s):
            def cg(t, acc):
                m = ck_v[pl.ds(t * 16, 16)] >= t_s
                return acc + plsc.all_reduce_population_count(m)
            return lax.fori_loop(0, nv, cg, zeros16)

        u = zeros16
        for b in range(31, -1, -1):
            bit = MIN32 if b == 31 else np.int32(1 << b)
            cand_u = u | bit
            cnt = count_ge(cand_u ^ MIN32)
            u = jnp.where(cnt >= k_rem, cand_u, u)
        thr = u ^ MIN32

        def cnt_gt(t, acc):
            m = ck_v[pl.ds(t * 16, 16)] > thr
            return acc + plsc.all_reduce_population_count(m)
        n_gt = lax.fori_loop(0, nv, cnt_gt, zeros16)
        need = k_rem - n_gt

        def cnt_eq_ge(jc):
            def ce(t, acc):
                m = (ck_v[pl.ds(t * 16, 16)] == thr) & (
                    ci_v[pl.ds(t * 16, 16)] >= jc)
                return acc + plsc.all_reduce_population_count(m)
            return lax.fori_loop(0, nv, ce, zeros16)

        n_eq = cnt_eq_ge(zeros16)
        jcut = zeros16
        for b in range(14, -1, -1):
            candj = jcut | np.int32(1 << b)
            jcut = jnp.where(cnt_eq_ge(candj) >= need, candj, jcut)
        jcut = jnp.where(n_eq == need, zeros16, jcut)

        def mk(i, _):
            x = row_v[pl.ds(i * 16, 16)]
            sk = _skey(x)
            keep = (sk > thr) | ((sk == thr) & (lane + i * 16 >= jcut))
            row_v[pl.ds(i * 16, 16)] = jnp.where(keep, x, 0.0)
            return 0
        lax.fori_loop(0, NV_ROW, mk, 0)
        pltpu.sync_copy(row_v, o_hbm.at[row])

    for j in range(ROWS_PER_WORKER):
        do_row(wid * ROWS_PER_WORKER + j)


@jax.jit
def kernel(s):
    mesh = plsc.VectorSubcoreMesh(core_axis_name="c", subcore_axis_name="s", num_cores=2, num_subcores=16)
    return pl.kernel(
        _sc_body,
        out_type=jax.ShapeDtypeStruct((BATCH_C, NEURONS_C), jnp.float32),
        mesh=mesh,
        compiler_params=pltpu.CompilerParams(needs_layout_passes=False),
        scratch_types=[
            pltpu.VMEM((NEURONS_C,), jnp.float32),
            pltpu.VMEM((4096,), jnp.int32),
            pltpu.VMEM((272,), jnp.int32),
            pltpu.VMEM((NEURONS_C + 16,), jnp.int32),
            pltpu.VMEM((NEURONS_C + 16,), jnp.int32),
        ],
    )(s)


# speedup vs baseline: 3.0589x; 3.0589x over previous
import numpy as np
import jax
import jax.numpy as jnp
from jax import lax
from jax.experimental import pallas as pl
from jax.experimental.pallas import tpu as pltpu, tpu_sc as plsc

NEURONS_C = 32768
K_C = 64
BATCH_C = 128
NWORKERS = 32
ROWS_PER_WORKER = BATCH_C // NWORKERS
NV_ROW = NEURONS_C // 16

MIN32 = np.int32(-2**31)
M7F = np.int32(0x7FFFFFFF)


def _skey(x):
    i = lax.bitcast_convert_type(x, jnp.int32)
    return i ^ (lax.shift_right_arithmetic(i, 31) & M7F)


def _sc_body(s_hbm, o_hbm, row_v, hist_v, suf_v, ck_v):
    wid = lax.axis_index("s") * 2 + lax.axis_index("c")
    lane = lax.iota(jnp.int32, 16)
    lane_base = lane * 256
    ones16 = jnp.ones((16,), jnp.int32)
    zeros16 = jnp.zeros((16,), jnp.int32)

    def do_row(row):
        pltpu.sync_copy(s_hbm.at[row], row_v)

        @plsc.parallel_loop(0, 256, unroll=8)
        def _zh(i):
            hist_v[pl.ds(i * 16, 16)] = zeros16

        @plsc.parallel_loop(0, NV_ROW, unroll=8)
        def _h(i):
            sk = _skey(row_v[pl.ds(i * 16, 16)])
            bin_ = lax.shift_right_logical(sk ^ MIN32, 24)
            plsc.addupdate_scatter(hist_v, [lane_base + bin_], ones16)

        suf_v[pl.ds(256, 16)] = zeros16
        running = jnp.int32(0)
        for c in range(15, -1, -1):
            tot = hist_v[pl.ds(c * 16, 16)]
            for j in range(1, 16):
                tot = tot + hist_v[pl.ds(j * 256 + c * 16, 16)]
            within = lax.rev(plsc.cumsum(lax.rev(tot, (0,))), (0,))
            suf_v[pl.ds(c * 16, 16)] = within + running
            running = running + jnp.sum(tot)

        b0 = zeros16
        for b in (128, 64, 32, 16, 8, 4, 2, 1):
            cand = b0 | np.int32(b)
            b0 = jnp.where(plsc.load_gather(suf_v, [cand]) >= K_C, cand, b0)
        c_above = plsc.load_gather(suf_v, [b0 + 1])
        k_rem = K_C - c_above
        n_cand = plsc.load_gather(suf_v, [b0]) - c_above

        @plsc.parallel_loop(0, NV_ROW, unroll=4, carry=zeros16)
        def off(i, off_c):
            sk = _skey(row_v[pl.ds(i * 16, 16)])
            bin_ = lax.shift_right_logical(sk ^ MIN32, 24)
            m = bin_ == b0
            pos = off_c + plsc.cumsum(m.astype(jnp.int32)) - 1
            plsc.store_scatter(ck_v, [pos], sk, mask=m)
            return off_c + plsc.all_reduce_population_count(m)

        for t in range(4):
            plsc.store_scatter(ck_v, [off + lane + t * 16],
                               jnp.full((16,), MIN32))
        nv4 = ((n_cand + 63) >> 6)[0]

        def count_ge(t_s):
            @plsc.parallel_loop(0, nv4, carry=zeros16)
            def acc(t, acc_c):
                for q in range(4):
                    m = ck_v[pl.ds(t * 64 + q * 16, 16)] >= t_s
                    acc_c = acc_c + plsc.all_reduce_population_count(m)
                return acc_c
            return acc

        u = lax.shift_left(b0, 24)
        for b in range(23, -1, -1):
            cand_u = u | np.int32(1 << b)
            cnt = count_ge(cand_u ^ MIN32)
            u = jnp.where(cnt >= k_rem, cand_u, u)
        thr = u ^ MIN32

        n_gt = count_ge(thr + 1)
        need = k_rem - n_gt
        n_eq = count_ge(thr) - n_gt

        def tie_search():
            jcut0 = zeros16
            for b in range(14, -1, -1):
                candj = jcut0 | np.int32(1 << b)

                @plsc.parallel_loop(0, NV_ROW, unroll=4, carry=zeros16)
                def cnt_j(i, c_c):
                    sk = _skey(row_v[pl.ds(i * 16, 16)])
                    m = (sk == thr) & (lane + i * 16 >= candj)
                    return c_c + plsc.all_reduce_population_count(m)
                jcut0 = jnp.where(cnt_j >= need, candj, jcut0)
            return jcut0

        jcut = lax.cond(((n_eq != need).astype(jnp.int32))[0] != 0, tie_search, lambda: zeros16)

        @plsc.parallel_loop(0, NV_ROW, unroll=8)
        def _mk(i):
            x = row_v[pl.ds(i * 16, 16)]
            sk = _skey(x)
            keep = (sk > thr) | ((sk == thr) & (lane + i * 16 >= jcut))
            row_v[pl.ds(i * 16, 16)] = jnp.where(keep, x, 0.0)

        pltpu.sync_copy(row_v, o_hbm.at[row])

    for j in range(ROWS_PER_WORKER):
        do_row(wid * ROWS_PER_WORKER + j)


@jax.jit
def kernel(s):
    mesh = plsc.VectorSubcoreMesh(core_axis_name="c", subcore_axis_name="s",
                                  num_cores=2, num_subcores=16)
    return pl.kernel(
        _sc_body,
        out_type=jax.ShapeDtypeStruct((BATCH_C, NEURONS_C), jnp.float32),
        mesh=mesh,
        compiler_params=pltpu.CompilerParams(needs_layout_passes=False),
        scratch_types=[
            pltpu.VMEM((NEURONS_C,), jnp.float32),
            pltpu.VMEM((4096,), jnp.int32),
            pltpu.VMEM((272,), jnp.int32),
            pltpu.VMEM((NEURONS_C + 64,), jnp.int32),
        ],
    )(s)
